# Initial kernel scaffold; baseline (speedup 1.0000x reference)
import functools

import jax
import jax.numpy as jnp
from jax import lax
from jax.experimental import pallas as pl
from jax.experimental.pallas import tpu as pltpu
from jax.experimental.pallas import tpu_sc as plsc

N = 10000
E = 320000
IN_D = 128
HID = 128
OUT_D = 64
NP = 10240
CHUNK = 128
NTILES = 32
NCH = 80
EP = NTILES * NCH * CHUNK
RPT = NP // 16
EPS = 1e-5


def _sc_mesh():
    return plsc.VectorSubcoreMesh(core_axis_name="c", subcore_axis_name="s")


@functools.partial(
    pl.kernel,
    out_type=jax.ShapeDtypeStruct((NTILES, NP), jnp.float32),
    mesh=_sc_mesh(),
    scratch_types=[
        pltpu.VMEM((NCH, CHUNK), jnp.int32),
        pltpu.VMEM((NP,), jnp.float32),
    ],
)
def _deg_kernel(dsts, out, dst_v, deg_v):
    c = lax.axis_index("c")
    s = lax.axis_index("s")
    wid = s * 2 + c
    zero = jnp.zeros((16,), jnp.float32)

    def zb(i, carry):
        deg_v[pl.ds(i * 16, 16)] = zero
        return carry

    lax.fori_loop(0, NP // 16, zb, 0)
    pltpu.sync_copy(dsts.at[wid], dst_v)
    ones = jnp.ones((16,), jnp.float32)

    def body(j, carry):
        def inner(t, carry2):
            idx = dst_v[j, pl.ds(t * 16, 16)]
            plsc.addupdate_scatter(deg_v, [idx], ones)
            return carry2

        return lax.fori_loop(0, CHUNK // 16, inner, carry)

    lax.fori_loop(0, NCH, body, 0)
    pltpu.sync_copy(deg_v, out.at[wid])


def _make_conv(D):
    @functools.partial(
        pl.kernel,
        out_type=jax.ShapeDtypeStruct((2, NP, D), jnp.float32),
        mesh=_sc_mesh(),
        scratch_types=[
            pltpu.VMEM((NCH, CHUNK), jnp.int32),
            pltpu.VMEM((NCH, CHUNK), jnp.int32),
            pltpu.VMEM((CHUNK, D), jnp.float32),
            pltpu.VMEM_SHARED((NP, D), jnp.float32),
            pltpu.SemaphoreType.DMA,
        ],
    )
    def conv_kernel(table, srcs, dsts, zrows, out, src_v, dst_v, rows_v,
                    acc_sh, sem):
        c = lax.axis_index("c")
        s = lax.axis_index("s")
        wid = s * 2 + c
        pltpu.sync_copy(zrows, acc_sh.at[pl.ds(s * RPT, RPT)])
        plsc.subcore_barrier()
        pltpu.sync_copy(srcs.at[wid], src_v)
        pltpu.sync_copy(dsts.at[wid], dst_v)

        def body(j, carry):
            pltpu.async_copy(table.at[src_v.at[j]], rows_v, sem).wait()
            pltpu.sync_copy(rows_v, acc_sh.at[dst_v.at[j]], add=True)
            return carry

        lax.fori_loop(0, NCH, body, 0)
        plsc.subcore_barrier()
        pltpu.sync_copy(acc_sh.at[pl.ds(s * RPT, RPT)],
                        out.at[c, pl.ds(s * RPT, RPT)])

    return conv_kernel


_conv128 = _make_conv(HID)
_conv64 = _make_conv(OUT_D)


def _stage0_body(degs_ref, x_ref, w_ref, dinv_ref, xt_ref):
    deg = jnp.sum(degs_ref[...], axis=0) + 1.0
    rows = lax.broadcasted_iota(jnp.int32, (NP, 1), 0)
    dinv = jnp.where(rows < N, lax.rsqrt(deg), 0.0)
    dinv_ref[...] = dinv
    xt_ref[...] = jnp.dot(x_ref[...], w_ref[...],
                          preferred_element_type=jnp.float32) * dinv


def _stage_mid_body(acc_ref, xt_ref, dinv_ref, b_ref, g_ref, be_ref, w_ref,
                    out_ref):
    dinv = dinv_ref[...]
    rows = lax.broadcasted_iota(jnp.int32, (NP, 1), 0)
    mask = (rows < N).astype(jnp.float32)
    h = ((acc_ref[0] + acc_ref[1] + xt_ref[...]) * dinv + b_ref[...]) * mask
    m = jnp.sum(h, axis=0, keepdims=True) * (1.0 / N)
    d = (h - m) * mask
    v = jnp.sum(d * d, axis=0, keepdims=True) * (1.0 / N)
    hb = d * lax.rsqrt(v + EPS) * g_ref[...] + be_ref[...]
    hr = jnp.maximum(hb, 0.0)
    out_ref[...] = jnp.dot(hr, w_ref[...],
                           preferred_element_type=jnp.float32) * dinv


def _stage3_body(acc_ref, xt_ref, dinv_ref, b_ref, w_ref, out_ref):
    dinv = dinv_ref[...]
    rows = lax.broadcasted_iota(jnp.int32, (NP, 1), 0)
    mask = (rows < N).astype(jnp.float32)
    h = ((acc_ref[0] + acc_ref[1] + xt_ref[...]) * dinv + b_ref[...]) * mask
    out_ref[...] = jnp.dot(h, w_ref[...],
                           preferred_element_type=jnp.float32) * dinv


def _stage4_body(acc_ref, xt_ref, dinv_ref, bf_ref, out_ref):
    dinv = dinv_ref[...]
    o = (acc_ref[0] + acc_ref[1] + xt_ref[...]) * dinv + bf_ref[...]
    o = o[:N]
    mx = jnp.max(o, axis=1, keepdims=True)
    lse = jnp.log(jnp.sum(jnp.exp(o - mx), axis=1, keepdims=True)) + mx
    out_ref[...] = o - lse


_stage0 = pl.pallas_call(
    _stage0_body,
    out_shape=(jax.ShapeDtypeStruct((NP, 1), jnp.float32),
               jax.ShapeDtypeStruct((NP, HID), jnp.float32)),
)

_stage_mid = pl.pallas_call(
    _stage_mid_body,
    out_shape=jax.ShapeDtypeStruct((NP, HID), jnp.float32),
)

_stage3 = pl.pallas_call(
    _stage3_body,
    out_shape=jax.ShapeDtypeStruct((NP, OUT_D), jnp.float32),
)

_stage4 = pl.pallas_call(
    _stage4_body,
    out_shape=jax.ShapeDtypeStruct((N, OUT_D), jnp.float32),
)


def kernel(x, edge_index, W0, b0, W1, b1, W2, b2, Wf, bf, g0, be0, g1, be1):
    pad = jnp.full((EP - E,), N, jnp.int32)
    srcs = jnp.concatenate([edge_index[0], pad]).reshape(NTILES, NCH, CHUNK)
    dsts = jnp.concatenate([edge_index[1], pad]).reshape(NTILES, NCH, CHUNK)
    xp = jnp.zeros((NP, IN_D), jnp.float32).at[:N].set(x)
    z128 = jnp.zeros((RPT, HID), jnp.float32)
    z64 = jnp.zeros((RPT, OUT_D), jnp.float32)
    b0r, b1r, b2r, bfr = (v.reshape(1, -1) for v in (b0, b1, b2, bf))
    g0r, be0r, g1r, be1r = (v.reshape(1, -1) for v in (g0, be0, g1, be1))

    deg_parts = _deg_kernel(dsts)
    dinv, xt0 = _stage0(deg_parts[..., None], xp, W0)
    acc = _conv128(xt0, srcs, dsts, z128)
    xt1 = _stage_mid(acc, xt0, dinv, b0r, g0r, be0r, W1)
    acc = _conv128(xt1, srcs, dsts, z128)
    xt2 = _stage_mid(acc, xt1, dinv, b1r, g1r, be1r, W2)
    acc = _conv128(xt2, srcs, dsts, z128)
    xt3 = _stage3(acc, xt2, dinv, b2r, Wf)
    acc4 = _conv64(xt3, srcs, dsts, z64)
    return _stage4(acc4, xt3, dinv, bfr)


# speedup vs baseline: 8.5417x; 8.5417x over previous
import functools

import jax
import jax.numpy as jnp
from jax import lax
from jax.experimental import pallas as pl
from jax.experimental.pallas import tpu as pltpu
from jax.experimental.pallas import tpu_sc as plsc

N = 10000
E = 320000
IN_D = 128
HID = 128
OUT_D = 64
NP = 10240
CHUNK = 128
NTILES = 32
NCH = 80
EP = NTILES * NCH * CHUNK
RPT = NP // 16
EPS = 1e-5


def _sc_mesh():
    return plsc.VectorSubcoreMesh(core_axis_name="c", subcore_axis_name="s")


@functools.partial(
    pl.kernel,
    out_type=jax.ShapeDtypeStruct((2, NP), jnp.float32),
    mesh=_sc_mesh(),
    scratch_types=[
        pltpu.VMEM((NCH, CHUNK), jnp.int32),
        pltpu.VMEM((CHUNK,), jnp.float32),
        pltpu.VMEM_SHARED((NP,), jnp.float32),
    ],
)
def _deg_kernel(dsts, ones, zer, out, dst_v, ones_v, deg_sh):
    c = lax.axis_index("c")
    s = lax.axis_index("s")
    wid = s * 2 + c
    pltpu.sync_copy(zer, deg_sh.at[pl.ds(s * RPT, RPT)])
    pltpu.sync_copy(ones, ones_v)
    plsc.subcore_barrier()
    pltpu.sync_copy(dsts.at[wid], dst_v)

    def body(j, carry):
        pltpu.sync_copy(ones_v, deg_sh.at[dst_v.at[j]], add=True)
        return carry

    lax.fori_loop(0, NCH, body, 0)
    plsc.subcore_barrier()
    pltpu.sync_copy(deg_sh.at[pl.ds(s * RPT, RPT)],
                    out.at[c, pl.ds(s * RPT, RPT)])


def _make_conv(D):
    @functools.partial(
        pl.kernel,
        out_type=jax.ShapeDtypeStruct((2, NP, D), jnp.float32),
        mesh=_sc_mesh(),
        scratch_types=[
            pltpu.VMEM((NCH, CHUNK), jnp.int32),
            pltpu.VMEM((NCH, CHUNK), jnp.int32),
            pltpu.VMEM((CHUNK, D), jnp.float32),
            pltpu.VMEM_SHARED((NP, D), jnp.float32),
            pltpu.SemaphoreType.DMA,
        ],
        compiler_params=pltpu.CompilerParams(use_tc_tiling_on_sc=False),
    )
    def conv_kernel(table, srcs, dsts, zrows, out, src_v, dst_v, rows_v,
                    acc_sh, sem):
        c = lax.axis_index("c")
        s = lax.axis_index("s")
        wid = s * 2 + c
        pltpu.sync_copy(zrows, acc_sh.at[pl.ds(s * RPT, RPT)])
        plsc.subcore_barrier()
        pltpu.sync_copy(srcs.at[wid], src_v)
        pltpu.sync_copy(dsts.at[wid], dst_v)

        def body(j, carry):
            pltpu.async_copy(table.at[src_v.at[j]], rows_v, sem).wait()
            pltpu.sync_copy(rows_v, acc_sh.at[dst_v.at[j]], add=True)
            return carry

        lax.fori_loop(0, NCH, body, 0)
        plsc.subcore_barrier()
        pltpu.sync_copy(acc_sh.at[pl.ds(s * RPT, RPT)],
                        out.at[c, pl.ds(s * RPT, RPT)])

    return conv_kernel


_conv128 = _make_conv(HID)
_conv64 = _make_conv(OUT_D)


def _stage0_body(degs_ref, x_ref, w_ref, dinv_ref, xt_ref):
    deg = jnp.sum(degs_ref[...], axis=0) + 1.0
    rows = lax.broadcasted_iota(jnp.int32, (NP, 1), 0)
    dinv = jnp.where(rows < N, lax.rsqrt(deg), 0.0)
    dinv_ref[...] = dinv
    xt_ref[...] = jnp.dot(x_ref[...], w_ref[...],
                          preferred_element_type=jnp.float32) * dinv


def _stage_mid_body(acc_ref, xt_ref, dinv_ref, b_ref, g_ref, be_ref, w_ref,
                    out_ref):
    dinv = dinv_ref[...]
    rows = lax.broadcasted_iota(jnp.int32, (NP, 1), 0)
    mask = (rows < N).astype(jnp.float32)
    h = ((acc_ref[0] + acc_ref[1] + xt_ref[...]) * dinv + b_ref[...]) * mask
    m = jnp.sum(h, axis=0, keepdims=True) * (1.0 / N)
    d = (h - m) * mask
    v = jnp.sum(d * d, axis=0, keepdims=True) * (1.0 / N)
    hb = d * lax.rsqrt(v + EPS) * g_ref[...] + be_ref[...]
    hr = jnp.maximum(hb, 0.0)
    out_ref[...] = jnp.dot(hr, w_ref[...],
                           preferred_element_type=jnp.float32) * dinv


def _stage3_body(acc_ref, xt_ref, dinv_ref, b_ref, w_ref, out_ref):
    dinv = dinv_ref[...]
    rows = lax.broadcasted_iota(jnp.int32, (NP, 1), 0)
    mask = (rows < N).astype(jnp.float32)
    h = ((acc_ref[0] + acc_ref[1] + xt_ref[...]) * dinv + b_ref[...]) * mask
    out_ref[...] = jnp.dot(h, w_ref[...],
                           preferred_element_type=jnp.float32) * dinv


def _stage4_body(acc_ref, xt_ref, dinv_ref, bf_ref, out_ref):
    dinv = dinv_ref[...]
    o = (acc_ref[0] + acc_ref[1] + xt_ref[...]) * dinv + bf_ref[...]
    o = o[:N]
    mx = jnp.max(o, axis=1, keepdims=True)
    lse = jnp.log(jnp.sum(jnp.exp(o - mx), axis=1, keepdims=True)) + mx
    out_ref[...] = o - lse


_stage0 = pl.pallas_call(
    _stage0_body,
    out_shape=(jax.ShapeDtypeStruct((NP, 1), jnp.float32),
               jax.ShapeDtypeStruct((NP, HID), jnp.float32)),
)

_stage_mid = pl.pallas_call(
    _stage_mid_body,
    out_shape=jax.ShapeDtypeStruct((NP, HID), jnp.float32),
)

_stage3 = pl.pallas_call(
    _stage3_body,
    out_shape=jax.ShapeDtypeStruct((NP, OUT_D), jnp.float32),
)

_stage4 = pl.pallas_call(
    _stage4_body,
    out_shape=jax.ShapeDtypeStruct((N, OUT_D), jnp.float32),
)


def kernel(x, edge_index, W0, b0, W1, b1, W2, b2, Wf, bf, g0, be0, g1, be1):
    pad = jnp.full((EP - E,), N, jnp.int32)
    srcs = jnp.concatenate([edge_index[0], pad]).reshape(NTILES, NCH, CHUNK)
    dsts = jnp.concatenate([edge_index[1], pad]).reshape(NTILES, NCH, CHUNK)
    xp = jnp.zeros((NP, IN_D), jnp.float32).at[:N].set(x)
    z128 = jnp.zeros((RPT, HID), jnp.float32)
    z64 = jnp.zeros((RPT, OUT_D), jnp.float32)
    b0r, b1r, b2r, bfr = (v.reshape(1, -1) for v in (b0, b1, b2, bf))
    g0r, be0r, g1r, be1r = (v.reshape(1, -1) for v in (g0, be0, g1, be1))

    ones_c = jnp.ones((CHUNK,), jnp.float32)
    zer_r = jnp.zeros((RPT,), jnp.float32)
    deg_parts = _deg_kernel(dsts, ones_c, zer_r)
    dinv, xt0 = _stage0(deg_parts[..., None], xp, W0)
    acc = _conv128(xt0, srcs, dsts, z128)
    xt1 = _stage_mid(acc, xt0, dinv, b0r, g0r, be0r, W1)
    acc = _conv128(xt1, srcs, dsts, z128)
    xt2 = _stage_mid(acc, xt1, dinv, b1r, g1r, be1r, W2)
    acc = _conv128(xt2, srcs, dsts, z128)
    xt3 = _stage3(acc, xt2, dinv, b2r, Wf)
    acc4 = _conv64(xt3, srcs, dsts, z64)
    return _stage4(acc4, xt3, dinv, bfr)
